# baseline (speedup 1.0000x reference)
import functools

import jax
import jax.numpy as jnp
from jax import lax
from jax.experimental import pallas as pl
from jax.experimental.pallas import tpu as pltpu
from jax.experimental.pallas import tpu_sc as plsc


def _gate_body(ea_ref, wg_ref, bg_ref, out_ref):
    z = jnp.dot(ea_ref[...], wg_ref[...], preferred_element_type=jnp.float32)
    out_ref[...] = jax.nn.sigmoid(z + bg_ref[...])


def _compute_gate(edge_attr, W_gate, b_gate):
    E, DE = edge_attr.shape
    D = W_gate.shape[1]
    EB = 4000
    return pl.pallas_call(
        _gate_body,
        grid=(E // EB,),
        in_specs=[
            pl.BlockSpec((EB, DE), lambda i: (i, 0)),
            pl.BlockSpec((DE, D), lambda i: (0, 0)),
            pl.BlockSpec((1, D), lambda i: (0, 0)),
        ],
        out_specs=pl.BlockSpec((EB, D), lambda i: (i, 0)),
        out_shape=jax.ShapeDtypeStruct((E, D), jnp.float32),
    )(edge_attr, W_gate, b_gate.reshape(1, D))


def _y2_body(x_ref, w_ref, b_ref, out_ref):
    out_ref[...] = (
        jnp.dot(x_ref[...], w_ref[...], preferred_element_type=jnp.float32)
        + b_ref[...]
    )


def _compute_y2(x, wep2, b_ep):
    N, D = x.shape
    EC = wep2.shape[1]
    w = jnp.pad(wep2, ((0, 0), (0, 16 - EC)))
    b = jnp.pad(b_ep, (0, 16 - EC)).reshape(1, 16)
    NB = 2000
    return pl.pallas_call(
        _y2_body,
        grid=(N // NB,),
        in_specs=[
            pl.BlockSpec((NB, D), lambda i: (i, 0)),
            pl.BlockSpec((D, 16), lambda i: (0, 0)),
            pl.BlockSpec((1, 16), lambda i: (0, 0)),
        ],
        out_specs=pl.BlockSpec((NB, 16), lambda i: (i, 0)),
        out_shape=jax.ShapeDtypeStruct((N, 16), jnp.float32),
    )(x, w, b)


def _vshuffle(x, idx16):
    return jax.lax.gather(
        x,
        idx16[:, None],
        jax.lax.GatherDimensionNumbers(
            offset_dims=(), collapsed_slice_dims=(0,), start_index_map=(0,)
        ),
        slice_sizes=(1,),
        mode=jax.lax.GatherScatterMode.PROMISE_IN_BOUNDS,
    )

_NCORES = 2
_NSUB = 16
_NW = _NCORES * _NSUB
_L = 16
_CHUNK = 80
_NBUF = 2


def _sc_edge_kernel(x, src, dst, gate, y2, wep1):
    N, D = x.shape
    E = src.shape[0]
    epw = E // _NW
    nch = epw // _CHUNK
    ngr = _CHUNK // _L
    nk = D // _L
    npad = ((N + _NSUB * 8 - 1) // (_NSUB * 8)) * (_NSUB * 8)
    rpt = npad // _NSUB
    ntr = (nch - 1) // _NBUF
    assert nch == _NBUF * ntr + 1

    mesh = plsc.VectorSubcoreMesh(
        core_axis_name="c", subcore_axis_name="s",
        num_cores=_NCORES, num_subcores=_NSUB,
    )

    @functools.partial(
        pl.kernel,
        out_type=[
            jax.ShapeDtypeStruct((2 * E,), jnp.float32),
            jax.ShapeDtypeStruct((_NCORES, npad, D), jnp.float32),
        ],
        mesh=mesh,
        compiler_params=pltpu.CompilerParams(use_tc_tiling_on_sc=False),
        scratch_types=[
            [pltpu.VMEM((_CHUNK,), jnp.int32)] * _NBUF,
            [pltpu.VMEM((_CHUNK,), jnp.int32)] * _NBUF,
            [pltpu.VMEM((_CHUNK, D), jnp.float32)] * _NBUF,
            [pltpu.VMEM((_CHUNK, D), jnp.float32)] * _NBUF,
            [pltpu.VMEM((2 * _CHUNK,), jnp.float32)] * _NBUF,
            [pltpu.VMEM((_CHUNK, 16), jnp.float32)] * _NBUF,
            pltpu.VMEM((2, D), jnp.float32),
            pltpu.VMEM_SHARED((npad, D), jnp.float32),
            [pltpu.SemaphoreType.DMA] * _NBUF,
            [pltpu.SemaphoreType.DMA] * _NBUF,
        ],
    )
    def run(x_hbm, src_hbm, dst_hbm, gate_hbm, y2_hbm, wep1_hbm, zeros_hbm,
            epf_hbm, aggp_hbm,
            sidx, didx, rows, gatev, epi, y2r, w_v, agg_sh, insem, outsem):
        cid = lax.axis_index("c")
        sid = lax.axis_index("s")
        wid = cid * _NSUB + sid

        pltpu.sync_copy(wep1_hbm, w_v)

        zf = jnp.zeros((_L,), jnp.float32)
        pltpu.sync_copy(zeros_hbm, agg_sh.at[pl.ds(sid * rpt, rpt)])
        plsc.subcore_barrier()

        iota16 = lax.iota(jnp.int32, _L)
        zi = jnp.zeros((_L,), jnp.int32)
        oi = zi + 1
        bfly = [iota16 ^ s for s in (8, 4, 2, 1)]
        half = iota16 >> 1
        halfb = half + 8
        even = (iota16 & 1) == 0
        wa = [w_v[0, pl.ds(k * _L, _L)] for k in range(nk)]
        wb = [w_v[1, pl.ds(k * _L, _L)] for k in range(nk)]

        def issue_ins(i, s):
            base = wid * epw + i * _CHUNK
            pltpu.sync_copy(src_hbm.at[pl.ds(base, _CHUNK)], sidx[s])
            pltpu.sync_copy(dst_hbm.at[pl.ds(base, _CHUNK)], didx[s])
            pltpu.async_copy(x_hbm.at[sidx[s]], rows[s], insem[s])
            pltpu.async_copy(y2_hbm.at[didx[s]], y2r[s], insem[s])
            pltpu.async_copy(gate_hbm.at[pl.ds(base, _CHUNK)], gatev[s],
                             insem[s])

        def drain_ins(i, s):
            base = wid * epw + i * _CHUNK
            pltpu.make_async_copy(x_hbm.at[sidx[s]], rows[s],
                                  insem[s]).wait()
            pltpu.make_async_copy(y2_hbm.at[didx[s]], y2r[s],
                                  insem[s]).wait()
            pltpu.make_async_copy(gate_hbm.at[pl.ds(base, _CHUNK)], gatev[s],
                                  insem[s]).wait()

        def issue_outs(i, s):
            base = wid * epw + i * _CHUNK
            pltpu.sync_copy(rows[s], agg_sh.at[didx[s]], add=True)
            pltpu.sync_copy(epi[s], epf_hbm.at[pl.ds(2 * base, 2 * _CHUNK)])

        def drain_outs(i, s):
            pass

        def compute(s):
            def group(g, _):
                ep0acc = zf
                ep1acc = zf
                for j in range(_L):
                    e = g * _L + j
                    acc0 = zf
                    acc1 = zf
                    for k in range(nk):
                        xv = rows[s][e, pl.ds(k * _L, _L)]
                        gv = gatev[s][e, pl.ds(k * _L, _L)]
                        mv = xv * gv
                        rows[s][e, pl.ds(k * _L, _L)] = mv
                        acc0 = acc0 + mv * wa[k]
                        acc1 = acc1 + mv * wb[k]
                    for st in bfly:
                        acc0 = acc0 + _vshuffle(acc0, st)
                        acc1 = acc1 + _vshuffle(acc1, st)
                    yrow = y2r[s][e, :]
                    sel = iota16 == j
                    ep0acc = jnp.where(sel, acc0 + _vshuffle(yrow, zi), ep0acc)
                    ep1acc = jnp.where(sel, acc1 + _vshuffle(yrow, oi), ep1acc)
                lo = jnp.where(even, _vshuffle(ep0acc, half),
                               _vshuffle(ep1acc, half))
                hi = jnp.where(even, _vshuffle(ep0acc, halfb),
                               _vshuffle(ep1acc, halfb))
                epi[s][pl.ds(g * 2 * _L, _L)] = lo
                epi[s][pl.ds(g * 2 * _L + _L, _L)] = hi
                return 0

            lax.fori_loop(0, ngr, group, 0)

        def phase(i, s):
            drain_ins(i, s)
            compute(s)
            issue_outs(i, s)

            @pl.when(i + _NBUF < nch)
            def _():
                issue_ins(i + _NBUF, s)

        issue_ins(0, 0)
        issue_ins(1, 1)

        def tbody(t, _):
            i0 = t * _NBUF
            phase(i0, 0)
            phase(i0 + 1, 1)
            return 0

        lax.fori_loop(0, ntr, tbody, 0)

        phase(nch - 1, (nch - 1) % _NBUF)

        plsc.subcore_barrier()

        pltpu.sync_copy(
            agg_sh.at[pl.ds(sid * rpt, rpt)],
            aggp_hbm.at[cid, pl.ds(sid * rpt, rpt)],
        )

    zeros = jnp.zeros((rpt, D), jnp.float32)
    return run(x, src, dst, gate, y2, wep1, zeros)


def _node_body(x_ref, a0_ref, a1_ref, wn1_ref, wn2_ref, bn_ref, wnp_ref,
               bnp_ref, out_ref):
    agg = a0_ref[...] + a1_ref[...]
    h = (
        jnp.dot(x_ref[...], wn1_ref[...], preferred_element_type=jnp.float32)
        + jnp.dot(agg, wn2_ref[...], preferred_element_type=jnp.float32)
        + bn_ref[...]
    )
    h = jnp.maximum(h, 0.0)
    out_ref[...] = (
        jnp.dot(h, wnp_ref[...], preferred_element_type=jnp.float32)
        + bnp_ref[...]
    )


def _node_mlp(x, a0, a1, wn1, wn2, b_node, W_np, b_np):
    N, D = x.shape
    H = wn1.shape[1]
    NCo = W_np.shape[1]
    NB = 2000
    return pl.pallas_call(
        _node_body,
        grid=(N // NB,),
        in_specs=[
            pl.BlockSpec((NB, D), lambda i: (i, 0)),
            pl.BlockSpec((NB, D), lambda i: (i, 0)),
            pl.BlockSpec((NB, D), lambda i: (i, 0)),
            pl.BlockSpec((D, H), lambda i: (0, 0)),
            pl.BlockSpec((D, H), lambda i: (0, 0)),
            pl.BlockSpec((1, H), lambda i: (0, 0)),
            pl.BlockSpec((H, NCo), lambda i: (0, 0)),
            pl.BlockSpec((1, NCo), lambda i: (0, 0)),
        ],
        out_specs=pl.BlockSpec((NB, NCo), lambda i: (i, 0)),
        out_shape=jax.ShapeDtypeStruct((N, NCo), jnp.float32),
    )(x, a0, a1, wn1, wn2, b_node.reshape(1, H), W_np, b_np.reshape(1, NCo))


def kernel(x, edge_index, edge_attr, batch, W_gate, b_gate, W_node, b_node,
           W_np, b_np, W_ep, b_ep):
    N, D = x.shape
    src = edge_index[0]
    dst = edge_index[1]

    gate = _compute_gate(edge_attr, W_gate, b_gate)
    wep2 = W_ep[D:]
    y2 = _compute_y2(x, wep2, b_ep)
    wep1 = W_ep[:D].T.reshape(2, D)

    epf, aggp = _sc_edge_kernel(x, src, dst, gate, y2, wep1)

    wn1 = W_node[:D]
    wn2 = W_node[D:]
    node_pred = _node_mlp(x, aggp[0], aggp[1], wn1, wn2, b_node, W_np, b_np)
    edge_pred = epf.reshape(src.shape[0], 2)
    return node_pred, edge_pred


# speedup vs baseline: 1.4671x; 1.4671x over previous
import functools

import jax
import jax.numpy as jnp
from jax import lax
from jax.experimental import pallas as pl
from jax.experimental.pallas import tpu as pltpu
from jax.experimental.pallas import tpu_sc as plsc


def _gate_body(ea_ref, wg_ref, bg_ref, out_ref):
    z = jnp.dot(ea_ref[...], wg_ref[...], preferred_element_type=jnp.float32)
    out_ref[...] = jax.nn.sigmoid(z + bg_ref[...])


def _compute_gate(edge_attr, W_gate, b_gate):
    E, DE = edge_attr.shape
    D = W_gate.shape[1]
    EB = 4000
    return pl.pallas_call(
        _gate_body,
        grid=(E // EB,),
        in_specs=[
            pl.BlockSpec((EB, DE), lambda i: (i, 0)),
            pl.BlockSpec((DE, D), lambda i: (0, 0)),
            pl.BlockSpec((1, D), lambda i: (0, 0)),
        ],
        out_specs=pl.BlockSpec((EB, D), lambda i: (i, 0)),
        out_shape=jax.ShapeDtypeStruct((E, D), jnp.float32),
    )(edge_attr, W_gate, b_gate.reshape(1, D))


def _y2_body(x_ref, w_ref, b_ref, out_ref):
    out_ref[...] = (
        jnp.dot(x_ref[...], w_ref[...], preferred_element_type=jnp.float32)
        + b_ref[...]
    )


def _compute_y2(x, wep2, b_ep):
    N, D = x.shape
    EC = wep2.shape[1]
    w = jnp.pad(wep2, ((0, 0), (0, 16 - EC)))
    b = jnp.pad(b_ep, (0, 16 - EC)).reshape(1, 16)
    NB = 2000
    return pl.pallas_call(
        _y2_body,
        grid=(N // NB,),
        in_specs=[
            pl.BlockSpec((NB, D), lambda i: (i, 0)),
            pl.BlockSpec((D, 16), lambda i: (0, 0)),
            pl.BlockSpec((1, 16), lambda i: (0, 0)),
        ],
        out_specs=pl.BlockSpec((NB, 16), lambda i: (i, 0)),
        out_shape=jax.ShapeDtypeStruct((N, 16), jnp.float32),
    )(x, w, b)


def _vshuffle(x, idx16):
    return jax.lax.gather(
        x,
        idx16[:, None],
        jax.lax.GatherDimensionNumbers(
            offset_dims=(), collapsed_slice_dims=(0,), start_index_map=(0,)
        ),
        slice_sizes=(1,),
        mode=jax.lax.GatherScatterMode.PROMISE_IN_BOUNDS,
    )

_NCORES = 2
_NSUB = 16
_NW = _NCORES * _NSUB
_L = 16
_CHUNK = 80
_NBUF = 2


def _sc_edge_kernel(x, src, dst, gate, y2, wep1):
    N, D = x.shape
    E = src.shape[0]
    epw = E // _NW
    nch = epw // _CHUNK
    ngr = _CHUNK // _L
    nk = D // _L
    npad = ((N + _NSUB * 8 - 1) // (_NSUB * 8)) * (_NSUB * 8)
    rpt = npad // _NSUB
    ntr = (nch - 1) // _NBUF
    assert nch == _NBUF * ntr + 1

    mesh = plsc.VectorSubcoreMesh(
        core_axis_name="c", subcore_axis_name="s",
        num_cores=_NCORES, num_subcores=_NSUB,
    )

    @functools.partial(
        pl.kernel,
        out_type=[
            jax.ShapeDtypeStruct((E,), jnp.float32),
            jax.ShapeDtypeStruct((E,), jnp.float32),
            jax.ShapeDtypeStruct((_NCORES, npad, D), jnp.float32),
        ],
        mesh=mesh,
        compiler_params=pltpu.CompilerParams(use_tc_tiling_on_sc=False),
        scratch_types=[
            [pltpu.VMEM((_CHUNK,), jnp.int32)] * _NBUF,
            [pltpu.VMEM((_CHUNK,), jnp.int32)] * _NBUF,
            [pltpu.VMEM((_CHUNK, D), jnp.float32)] * _NBUF,
            [pltpu.VMEM((_CHUNK, D), jnp.float32)] * _NBUF,
            [pltpu.VMEM((_CHUNK,), jnp.float32)] * _NBUF,
            [pltpu.VMEM((_CHUNK,), jnp.float32)] * _NBUF,
            [pltpu.VMEM((_CHUNK, 16), jnp.float32)] * _NBUF,
            pltpu.VMEM((2, D), jnp.float32),
            pltpu.VMEM_SHARED((npad, D), jnp.float32),
            [pltpu.SemaphoreType.DMA] * _NBUF,
            [pltpu.SemaphoreType.DMA] * _NBUF,
        ],
    )
    def run(x_hbm, src_hbm, dst_hbm, gate_hbm, y2_hbm, wep1_hbm, zeros_hbm,
            ep0_hbm, ep1_hbm, aggp_hbm,
            sidx, didx, rows, gatev, ep0, ep1, y2r, w_v, agg_sh,
            insem, outsem):
        cid = lax.axis_index("c")
        sid = lax.axis_index("s")
        wid = cid * _NSUB + sid

        pltpu.sync_copy(wep1_hbm, w_v)

        zf = jnp.zeros((_L,), jnp.float32)
        pltpu.sync_copy(zeros_hbm, agg_sh.at[pl.ds(sid * rpt, rpt)])
        plsc.subcore_barrier()

        iota16 = lax.iota(jnp.int32, _L)
        zi = jnp.zeros((_L,), jnp.int32)
        oi = zi + 1
        bfly = [iota16 ^ s for s in (8, 4, 2, 1)]
        wa = [w_v[0, pl.ds(k * _L, _L)] for k in range(nk)]
        wb = [w_v[1, pl.ds(k * _L, _L)] for k in range(nk)]

        def issue_ins(i, s):
            base = wid * epw + i * _CHUNK
            pltpu.sync_copy(src_hbm.at[pl.ds(base, _CHUNK)], sidx[s])
            pltpu.sync_copy(dst_hbm.at[pl.ds(base, _CHUNK)], didx[s])
            pltpu.async_copy(x_hbm.at[sidx[s]], rows[s], insem[s])
            pltpu.async_copy(y2_hbm.at[didx[s]], y2r[s], insem[s])
            pltpu.async_copy(gate_hbm.at[pl.ds(base, _CHUNK)], gatev[s],
                             insem[s])

        def drain_ins(i, s):
            base = wid * epw + i * _CHUNK
            pltpu.make_async_copy(x_hbm.at[sidx[s]], rows[s],
                                  insem[s]).wait()
            pltpu.make_async_copy(y2_hbm.at[didx[s]], y2r[s],
                                  insem[s]).wait()
            pltpu.make_async_copy(gate_hbm.at[pl.ds(base, _CHUNK)], gatev[s],
                                  insem[s]).wait()

        def issue_outs(i, s):
            base = wid * epw + i * _CHUNK
            pltpu.sync_copy(rows[s], agg_sh.at[didx[s]], add=True)
            pltpu.sync_copy(ep0[s], ep0_hbm.at[pl.ds(base, _CHUNK)])
            pltpu.sync_copy(ep1[s], ep1_hbm.at[pl.ds(base, _CHUNK)])

        def compute(s):
            def group(g, _):
                ep0acc = zf
                ep1acc = zf
                for j in range(_L):
                    e = g * _L + j
                    acc0 = zf
                    acc1 = zf
                    for k in range(nk):
                        xv = rows[s][e, pl.ds(k * _L, _L)]
                        gv = gatev[s][e, pl.ds(k * _L, _L)]
                        mv = xv * gv
                        rows[s][e, pl.ds(k * _L, _L)] = mv
                        acc0 = acc0 + mv * wa[k]
                        acc1 = acc1 + mv * wb[k]
                    for st in bfly:
                        acc0 = acc0 + _vshuffle(acc0, st)
                        acc1 = acc1 + _vshuffle(acc1, st)
                    yrow = y2r[s][e, :]
                    sel = iota16 == j
                    ep0acc = jnp.where(sel, acc0 + _vshuffle(yrow, zi), ep0acc)
                    ep1acc = jnp.where(sel, acc1 + _vshuffle(yrow, oi), ep1acc)
                ep0[s][pl.ds(g * _L, _L)] = ep0acc
                ep1[s][pl.ds(g * _L, _L)] = ep1acc
                return 0

            lax.fori_loop(0, ngr, group, 0)

        def phase(i, s):
            drain_ins(i, s)
            compute(s)
            issue_outs(i, s)

            @pl.when(i + _NBUF < nch)
            def _():
                issue_ins(i + _NBUF, s)

        issue_ins(0, 0)
        issue_ins(1, 1)

        def tbody(t, _):
            i0 = t * _NBUF
            phase(i0, 0)
            phase(i0 + 1, 1)
            return 0

        lax.fori_loop(0, ntr, tbody, 0)

        phase(nch - 1, (nch - 1) % _NBUF)

        plsc.subcore_barrier()

        pltpu.sync_copy(
            agg_sh.at[pl.ds(sid * rpt, rpt)],
            aggp_hbm.at[cid, pl.ds(sid * rpt, rpt)],
        )

    zeros = jnp.zeros((rpt, D), jnp.float32)
    return run(x, src, dst, gate, y2, wep1, zeros)


def _node_body(x_ref, a0_ref, a1_ref, wn1_ref, wn2_ref, bn_ref, wnp_ref,
               bnp_ref, out_ref):
    agg = a0_ref[...] + a1_ref[...]
    h = (
        jnp.dot(x_ref[...], wn1_ref[...], preferred_element_type=jnp.float32)
        + jnp.dot(agg, wn2_ref[...], preferred_element_type=jnp.float32)
        + bn_ref[...]
    )
    h = jnp.maximum(h, 0.0)
    out_ref[...] = (
        jnp.dot(h, wnp_ref[...], preferred_element_type=jnp.float32)
        + bnp_ref[...]
    )


def _node_mlp(x, a0, a1, wn1, wn2, b_node, W_np, b_np):
    N, D = x.shape
    H = wn1.shape[1]
    NCo = W_np.shape[1]
    NB = 2000
    return pl.pallas_call(
        _node_body,
        grid=(N // NB,),
        in_specs=[
            pl.BlockSpec((NB, D), lambda i: (i, 0)),
            pl.BlockSpec((NB, D), lambda i: (i, 0)),
            pl.BlockSpec((NB, D), lambda i: (i, 0)),
            pl.BlockSpec((D, H), lambda i: (0, 0)),
            pl.BlockSpec((D, H), lambda i: (0, 0)),
            pl.BlockSpec((1, H), lambda i: (0, 0)),
            pl.BlockSpec((H, NCo), lambda i: (0, 0)),
            pl.BlockSpec((1, NCo), lambda i: (0, 0)),
        ],
        out_specs=pl.BlockSpec((NB, NCo), lambda i: (i, 0)),
        out_shape=jax.ShapeDtypeStruct((N, NCo), jnp.float32),
    )(x, a0, a1, wn1, wn2, b_node.reshape(1, H), W_np, b_np.reshape(1, NCo))


def kernel(x, edge_index, edge_attr, batch, W_gate, b_gate, W_node, b_node,
           W_np, b_np, W_ep, b_ep):
    N, D = x.shape
    src = edge_index[0]
    dst = edge_index[1]

    gate = _compute_gate(edge_attr, W_gate, b_gate)
    wep2 = W_ep[D:]
    y2 = _compute_y2(x, wep2, b_ep)
    wep1 = W_ep[:D].T.reshape(2, D)

    ep0v, ep1v, aggp = _sc_edge_kernel(x, src, dst, gate, y2, wep1)

    wn1 = W_node[:D]
    wn2 = W_node[D:]
    node_pred = _node_mlp(x, aggp[0], aggp[1], wn1, wn2, b_node, W_np, b_np)
    edge_pred = jnp.stack([ep0v, ep1v], axis=1)
    return node_pred, edge_pred


# speedup vs baseline: 1.4901x; 1.0157x over previous
import functools

import jax
import jax.numpy as jnp
from jax import lax
from jax.experimental import pallas as pl
from jax.experimental.pallas import tpu as pltpu
from jax.experimental.pallas import tpu_sc as plsc


def _gate_body(ea_ref, wg_ref, bg_ref, out_ref):
    z = jnp.dot(ea_ref[...], wg_ref[...], preferred_element_type=jnp.float32)
    out_ref[...] = jax.nn.sigmoid(z + bg_ref[...])


def _compute_gate(edge_attr, W_gate, b_gate):
    E, DE = edge_attr.shape
    D = W_gate.shape[1]
    EB = 4000
    return pl.pallas_call(
        _gate_body,
        grid=(E // EB,),
        in_specs=[
            pl.BlockSpec((EB, DE), lambda i: (i, 0)),
            pl.BlockSpec((DE, D), lambda i: (0, 0)),
            pl.BlockSpec((1, D), lambda i: (0, 0)),
        ],
        out_specs=pl.BlockSpec((EB, D), lambda i: (i, 0)),
        out_shape=jax.ShapeDtypeStruct((E, D), jnp.float32),
    )(edge_attr, W_gate, b_gate.reshape(1, D))


def _y2_body(x_ref, w_ref, b_ref, out_ref):
    out_ref[...] = (
        jnp.dot(x_ref[...], w_ref[...], preferred_element_type=jnp.float32)
        + b_ref[...]
    )


def _compute_y2(x, wep2, b_ep):
    N, D = x.shape
    EC = wep2.shape[1]
    w = jnp.pad(wep2, ((0, 0), (0, 16 - EC)))
    b = jnp.pad(b_ep, (0, 16 - EC)).reshape(1, 16)
    NB = 2000
    return pl.pallas_call(
        _y2_body,
        grid=(N // NB,),
        in_specs=[
            pl.BlockSpec((NB, D), lambda i: (i, 0)),
            pl.BlockSpec((D, 16), lambda i: (0, 0)),
            pl.BlockSpec((1, 16), lambda i: (0, 0)),
        ],
        out_specs=pl.BlockSpec((NB, 16), lambda i: (i, 0)),
        out_shape=jax.ShapeDtypeStruct((N, 16), jnp.float32),
    )(x, w, b)


def _vshuffle(x, idx16):
    return jax.lax.gather(
        x,
        idx16[:, None],
        jax.lax.GatherDimensionNumbers(
            offset_dims=(), collapsed_slice_dims=(0,), start_index_map=(0,)
        ),
        slice_sizes=(1,),
        mode=jax.lax.GatherScatterMode.PROMISE_IN_BOUNDS,
    )

_NCORES = 2
_NSUB = 16
_NW = _NCORES * _NSUB
_L = 16
_CHUNK = 80
_NBUF = 2


def _sc_edge_kernel(x, src, dst, gate, y2, wep1):
    N, D = x.shape
    E = src.shape[0]
    epw = E // _NW
    nch = epw // _CHUNK
    ngr = _CHUNK // _L
    nk = D // _L
    npad = ((N + _NSUB * 8 - 1) // (_NSUB * 8)) * (_NSUB * 8)
    rpt = npad // _NSUB
    ntr = (nch - 1) // _NBUF
    assert nch == _NBUF * ntr + 1
    fl = 25
    assert nch % fl == 0

    mesh = plsc.VectorSubcoreMesh(
        core_axis_name="c", subcore_axis_name="s",
        num_cores=_NCORES, num_subcores=_NSUB,
    )

    @functools.partial(
        pl.kernel,
        out_type=[
            jax.ShapeDtypeStruct((E,), jnp.float32),
            jax.ShapeDtypeStruct((E,), jnp.float32),
            jax.ShapeDtypeStruct((_NCORES, npad, D), jnp.float32),
        ],
        mesh=mesh,
        compiler_params=pltpu.CompilerParams(use_tc_tiling_on_sc=False),
        scratch_types=[
            [pltpu.VMEM((_CHUNK,), jnp.int32)] * _NBUF,
            [pltpu.VMEM((_CHUNK,), jnp.int32)] * _NBUF,
            [pltpu.VMEM((_CHUNK, D), jnp.float32)] * _NBUF,
            [pltpu.VMEM((_CHUNK, D), jnp.float32)] * _NBUF,
            pltpu.VMEM((fl * _CHUNK,), jnp.float32),
            pltpu.VMEM((fl * _CHUNK,), jnp.float32),
            [pltpu.VMEM((_CHUNK, 16), jnp.float32)] * _NBUF,
            pltpu.VMEM((2, D), jnp.float32),
            pltpu.VMEM_SHARED((npad, D), jnp.float32),
            [pltpu.SemaphoreType.DMA] * _NBUF,
            [pltpu.SemaphoreType.DMA] * _NBUF,
        ],
    )
    def run(x_hbm, src_hbm, dst_hbm, gate_hbm, y2_hbm, wep1_hbm, zeros_hbm,
            ep0_hbm, ep1_hbm, aggp_hbm,
            sidx, didx, rows, gatev, ep0, ep1, y2r, w_v, agg_sh,
            insem, outsem):
        cid = lax.axis_index("c")
        sid = lax.axis_index("s")
        wid = cid * _NSUB + sid

        pltpu.sync_copy(wep1_hbm, w_v)

        zf = jnp.zeros((_L,), jnp.float32)
        pltpu.sync_copy(zeros_hbm, agg_sh.at[pl.ds(sid * rpt, rpt)])
        plsc.subcore_barrier()

        iota16 = lax.iota(jnp.int32, _L)
        zi = jnp.zeros((_L,), jnp.int32)
        oi = zi + 1
        bfly = [iota16 ^ s for s in (8, 4, 2, 1)]
        wa = [w_v[0, pl.ds(k * _L, _L)] for k in range(nk)]
        wb = [w_v[1, pl.ds(k * _L, _L)] for k in range(nk)]

        def issue_ins(i, s):
            base = wid * epw + i * _CHUNK
            pltpu.sync_copy(src_hbm.at[pl.ds(base, _CHUNK)], sidx[s])
            pltpu.sync_copy(dst_hbm.at[pl.ds(base, _CHUNK)], didx[s])
            pltpu.async_copy(x_hbm.at[sidx[s]], rows[s], insem[s])
            pltpu.async_copy(y2_hbm.at[didx[s]], y2r[s], insem[s])
            pltpu.async_copy(gate_hbm.at[pl.ds(base, _CHUNK)], gatev[s],
                             insem[s])

        def drain_ins(i, s):
            base = wid * epw + i * _CHUNK
            pltpu.make_async_copy(x_hbm.at[sidx[s]], rows[s],
                                  insem[s]).wait()
            pltpu.make_async_copy(y2_hbm.at[didx[s]], y2r[s],
                                  insem[s]).wait()
            pltpu.make_async_copy(gate_hbm.at[pl.ds(base, _CHUNK)], gatev[s],
                                  insem[s]).wait()

        def issue_outs(i, s):
            pltpu.sync_copy(rows[s], agg_sh.at[didx[s]], add=True)

            @pl.when(lax.rem(i, fl) == fl - 1)
            def _():
                fbase = wid * epw + (i - (fl - 1)) * _CHUNK
                pltpu.sync_copy(ep0, ep0_hbm.at[pl.ds(fbase, fl * _CHUNK)])
                pltpu.sync_copy(ep1, ep1_hbm.at[pl.ds(fbase, fl * _CHUNK)])

        def compute(i, s):
            eoff = lax.rem(i, fl) * _CHUNK

            def group(g, _):
                ep0acc = zf
                ep1acc = zf
                for j in range(_L):
                    e = g * _L + j
                    acc0 = zf
                    acc1 = zf
                    for k in range(nk):
                        xv = rows[s][e, pl.ds(k * _L, _L)]
                        gv = gatev[s][e, pl.ds(k * _L, _L)]
                        mv = xv * gv
                        rows[s][e, pl.ds(k * _L, _L)] = mv
                        acc0 = acc0 + mv * wa[k]
                        acc1 = acc1 + mv * wb[k]
                    for st in bfly:
                        acc0 = acc0 + _vshuffle(acc0, st)
                        acc1 = acc1 + _vshuffle(acc1, st)
                    yrow = y2r[s][e, :]
                    sel = iota16 == j
                    ep0acc = jnp.where(sel, acc0 + _vshuffle(yrow, zi), ep0acc)
                    ep1acc = jnp.where(sel, acc1 + _vshuffle(yrow, oi), ep1acc)
                ep0[pl.ds(eoff + g * _L, _L)] = ep0acc
                ep1[pl.ds(eoff + g * _L, _L)] = ep1acc
                return 0

            lax.fori_loop(0, ngr, group, 0)

        def phase(i, s):
            drain_ins(i, s)
            compute(i, s)
            issue_outs(i, s)

            @pl.when(i + _NBUF < nch)
            def _():
                issue_ins(i + _NBUF, s)

        issue_ins(0, 0)
        issue_ins(1, 1)

        def tbody(t, _):
            i0 = t * _NBUF
            phase(i0, 0)
            phase(i0 + 1, 1)
            return 0

        lax.fori_loop(0, ntr, tbody, 0)

        phase(nch - 1, (nch - 1) % _NBUF)

        plsc.subcore_barrier()

        pltpu.sync_copy(
            agg_sh.at[pl.ds(sid * rpt, rpt)],
            aggp_hbm.at[cid, pl.ds(sid * rpt, rpt)],
        )

    zeros = jnp.zeros((rpt, D), jnp.float32)
    return run(x, src, dst, gate, y2, wep1, zeros)


def _node_body(x_ref, a0_ref, a1_ref, wn1_ref, wn2_ref, bn_ref, wnp_ref,
               bnp_ref, out_ref):
    agg = a0_ref[...] + a1_ref[...]
    h = (
        jnp.dot(x_ref[...], wn1_ref[...], preferred_element_type=jnp.float32)
        + jnp.dot(agg, wn2_ref[...], preferred_element_type=jnp.float32)
        + bn_ref[...]
    )
    h = jnp.maximum(h, 0.0)
    out_ref[...] = (
        jnp.dot(h, wnp_ref[...], preferred_element_type=jnp.float32)
        + bnp_ref[...]
    )


def _node_mlp(x, a0, a1, wn1, wn2, b_node, W_np, b_np):
    N, D = x.shape
    H = wn1.shape[1]
    NCo = W_np.shape[1]
    NB = 2000
    return pl.pallas_call(
        _node_body,
        grid=(N // NB,),
        in_specs=[
            pl.BlockSpec((NB, D), lambda i: (i, 0)),
            pl.BlockSpec((NB, D), lambda i: (i, 0)),
            pl.BlockSpec((NB, D), lambda i: (i, 0)),
            pl.BlockSpec((D, H), lambda i: (0, 0)),
            pl.BlockSpec((D, H), lambda i: (0, 0)),
            pl.BlockSpec((1, H), lambda i: (0, 0)),
            pl.BlockSpec((H, NCo), lambda i: (0, 0)),
            pl.BlockSpec((1, NCo), lambda i: (0, 0)),
        ],
        out_specs=pl.BlockSpec((NB, NCo), lambda i: (i, 0)),
        out_shape=jax.ShapeDtypeStruct((N, NCo), jnp.float32),
    )(x, a0, a1, wn1, wn2, b_node.reshape(1, H), W_np, b_np.reshape(1, NCo))


def kernel(x, edge_index, edge_attr, batch, W_gate, b_gate, W_node, b_node,
           W_np, b_np, W_ep, b_ep):
    N, D = x.shape
    src = edge_index[0]
    dst = edge_index[1]

    gate = _compute_gate(edge_attr, W_gate, b_gate)
    wep2 = W_ep[D:]
    y2 = _compute_y2(x, wep2, b_ep)
    wep1 = W_ep[:D].T.reshape(2, D)

    ep0v, ep1v, aggp = _sc_edge_kernel(x, src, dst, gate, y2, wep1)

    wn1 = W_node[:D]
    wn2 = W_node[D:]
    node_pred = _node_mlp(x, aggp[0], aggp[1], wn1, wn2, b_node, W_np, b_np)
    edge_pred = jnp.stack([ep0v, ep1v], axis=1)
    return node_pred, edge_pred


# speedup vs baseline: 1.5702x; 1.0537x over previous
import functools

import jax
import jax.numpy as jnp
from jax import lax
from jax.experimental import pallas as pl
from jax.experimental.pallas import tpu as pltpu
from jax.experimental.pallas import tpu_sc as plsc


def _gate_body(ea_ref, wg_ref, bg_ref, out_ref):
    z = jnp.dot(ea_ref[...], wg_ref[...], preferred_element_type=jnp.float32)
    out_ref[...] = jax.nn.sigmoid(z + bg_ref[...])


def _compute_gate(edge_attr, W_gate, b_gate):
    E, DE = edge_attr.shape
    D = W_gate.shape[1]
    EB = 4000
    return pl.pallas_call(
        _gate_body,
        grid=(E // EB,),
        in_specs=[
            pl.BlockSpec((EB, DE), lambda i: (i, 0)),
            pl.BlockSpec((DE, D), lambda i: (0, 0)),
            pl.BlockSpec((1, D), lambda i: (0, 0)),
        ],
        out_specs=pl.BlockSpec((EB, D), lambda i: (i, 0)),
        out_shape=jax.ShapeDtypeStruct((E, D), jnp.float32),
    )(edge_attr, W_gate, b_gate.reshape(1, D))


def _y2_body(x_ref, w_ref, b_ref, out_ref):
    out_ref[...] = (
        jnp.dot(x_ref[...], w_ref[...], preferred_element_type=jnp.float32)
        + b_ref[...]
    )


def _compute_y2(x, wep2, b_ep):
    N, D = x.shape
    EC = wep2.shape[1]
    w = jnp.pad(wep2, ((0, 0), (0, 16 - EC)))
    b = jnp.pad(b_ep, (0, 16 - EC)).reshape(1, 16)
    NB = 2000
    return pl.pallas_call(
        _y2_body,
        grid=(N // NB,),
        in_specs=[
            pl.BlockSpec((NB, D), lambda i: (i, 0)),
            pl.BlockSpec((D, 16), lambda i: (0, 0)),
            pl.BlockSpec((1, 16), lambda i: (0, 0)),
        ],
        out_specs=pl.BlockSpec((NB, 16), lambda i: (i, 0)),
        out_shape=jax.ShapeDtypeStruct((N, 16), jnp.float32),
    )(x, w, b)


def _vshuffle(x, idx16):
    return jax.lax.gather(
        x,
        idx16[:, None],
        jax.lax.GatherDimensionNumbers(
            offset_dims=(), collapsed_slice_dims=(0,), start_index_map=(0,)
        ),
        slice_sizes=(1,),
        mode=jax.lax.GatherScatterMode.PROMISE_IN_BOUNDS,
    )

_NCORES = 2
_NSUB = 16
_NW = _NCORES * _NSUB
_L = 16
_CHUNK = 80
_NBUF = 2


def _sc_edge_kernel(x, src, dst, gate, y2, wep1):
    N, D = x.shape
    E = src.shape[0]
    epw = E // _NW
    nch = epw // _CHUNK
    ngr = _CHUNK // _L
    nk = D // _L
    npad = ((N + _NSUB * 8 - 1) // (_NSUB * 8)) * (_NSUB * 8)
    rpt = npad // _NSUB
    ntr = (nch - 1) // _NBUF
    assert nch == _NBUF * ntr + 1
    fl = 25
    assert nch % fl == 0

    mesh = plsc.VectorSubcoreMesh(
        core_axis_name="c", subcore_axis_name="s",
        num_cores=_NCORES, num_subcores=_NSUB,
    )

    @functools.partial(
        pl.kernel,
        out_type=[
            jax.ShapeDtypeStruct((E,), jnp.float32),
            jax.ShapeDtypeStruct((E,), jnp.float32),
            jax.ShapeDtypeStruct((_NCORES, npad, D), jnp.float32),
        ],
        mesh=mesh,
        compiler_params=pltpu.CompilerParams(use_tc_tiling_on_sc=False),
        scratch_types=[
            [pltpu.VMEM((2, _CHUNK), jnp.int32)] * _NBUF,
            [pltpu.VMEM((_CHUNK, D), jnp.float32)] * _NBUF,
            [pltpu.VMEM((_CHUNK, D), jnp.float32)] * _NBUF,
            pltpu.VMEM((fl * _CHUNK,), jnp.float32),
            pltpu.VMEM((fl * _CHUNK,), jnp.float32),
            [pltpu.VMEM((_CHUNK, 16), jnp.float32)] * _NBUF,
            pltpu.VMEM((2, D), jnp.float32),
            pltpu.VMEM_SHARED((npad, D), jnp.float32),
            [pltpu.SemaphoreType.DMA] * _NBUF,
            [pltpu.SemaphoreType.DMA] * _NBUF,
        ],
    )
    def run(x_hbm, sd_hbm, gate_hbm, y2_hbm, wep1_hbm, zeros_hbm,
            ep0_hbm, ep1_hbm, aggp_hbm,
            sdidx, rows, gatev, ep0, ep1, y2r, w_v, agg_sh,
            insem, outsem):
        cid = lax.axis_index("c")
        sid = lax.axis_index("s")
        wid = cid * _NSUB + sid

        pltpu.sync_copy(wep1_hbm, w_v)

        zf = jnp.zeros((_L,), jnp.float32)
        pltpu.sync_copy(zeros_hbm, agg_sh.at[pl.ds(sid * rpt, rpt)])
        plsc.subcore_barrier()

        iota16 = lax.iota(jnp.int32, _L)
        zi = jnp.zeros((_L,), jnp.int32)
        oi = zi + 1
        bfly = [iota16 ^ s for s in (8, 4, 2, 1)]
        wa = [w_v[0, pl.ds(k * _L, _L)] for k in range(nk)]
        wb = [w_v[1, pl.ds(k * _L, _L)] for k in range(nk)]

        def issue_ins(i, s):
            base = wid * epw + i * _CHUNK
            pltpu.sync_copy(sd_hbm.at[wid * nch + i], sdidx[s])
            pltpu.async_copy(x_hbm.at[sdidx[s].at[0]], rows[s], insem[s])
            pltpu.async_copy(y2_hbm.at[sdidx[s].at[1]], y2r[s], insem[s])
            pltpu.async_copy(gate_hbm.at[pl.ds(base, _CHUNK)], gatev[s],
                             insem[s])

        def drain_ins(i, s):
            base = wid * epw + i * _CHUNK
            pltpu.make_async_copy(x_hbm.at[sdidx[s].at[0]], rows[s],
                                  insem[s]).wait()
            pltpu.make_async_copy(y2_hbm.at[sdidx[s].at[1]], y2r[s],
                                  insem[s]).wait()
            pltpu.make_async_copy(gate_hbm.at[pl.ds(base, _CHUNK)], gatev[s],
                                  insem[s]).wait()

        def issue_outs(i, s):
            pltpu.sync_copy(rows[s], agg_sh.at[sdidx[s].at[1]], add=True)

            @pl.when(lax.rem(i, fl) == fl - 1)
            def _():
                fbase = wid * epw + (i - (fl - 1)) * _CHUNK
                pltpu.sync_copy(ep0, ep0_hbm.at[pl.ds(fbase, fl * _CHUNK)])
                pltpu.sync_copy(ep1, ep1_hbm.at[pl.ds(fbase, fl * _CHUNK)])

        def compute(i, s):
            eoff = lax.rem(i, fl) * _CHUNK

            def group(g, _):
                ep0acc = zf
                ep1acc = zf
                for j in range(_L):
                    e = g * _L + j
                    acc0 = zf
                    acc1 = zf
                    for k in range(nk):
                        xv = rows[s][e, pl.ds(k * _L, _L)]
                        gv = gatev[s][e, pl.ds(k * _L, _L)]
                        mv = xv * gv
                        rows[s][e, pl.ds(k * _L, _L)] = mv
                        acc0 = acc0 + mv * wa[k]
                        acc1 = acc1 + mv * wb[k]
                    for st in bfly:
                        acc0 = acc0 + _vshuffle(acc0, st)
                        acc1 = acc1 + _vshuffle(acc1, st)
                    yrow = y2r[s][e, :]
                    sel = iota16 == j
                    ep0acc = jnp.where(sel, acc0 + _vshuffle(yrow, zi), ep0acc)
                    ep1acc = jnp.where(sel, acc1 + _vshuffle(yrow, oi), ep1acc)
                ep0[pl.ds(eoff + g * _L, _L)] = ep0acc
                ep1[pl.ds(eoff + g * _L, _L)] = ep1acc
                return 0

            lax.fori_loop(0, ngr, group, 0)

        def phase(i, s):
            drain_ins(i, s)
            compute(i, s)
            issue_outs(i, s)

            @pl.when(i + _NBUF < nch)
            def _():
                issue_ins(i + _NBUF, s)

        issue_ins(0, 0)
        issue_ins(1, 1)

        def tbody(t, _):
            i0 = t * _NBUF
            phase(i0, 0)
            phase(i0 + 1, 1)
            return 0

        lax.fori_loop(0, ntr, tbody, 0)

        phase(nch - 1, (nch - 1) % _NBUF)

        plsc.subcore_barrier()

        pltpu.sync_copy(
            agg_sh.at[pl.ds(sid * rpt, rpt)],
            aggp_hbm.at[cid, pl.ds(sid * rpt, rpt)],
        )

    sd = jnp.stack(
        [src.reshape(_NW * nch, _CHUNK), dst.reshape(_NW * nch, _CHUNK)],
        axis=1,
    )
    zeros = jnp.zeros((rpt, D), jnp.float32)
    return run(x, sd, gate, y2, wep1, zeros)


def _node_body(x_ref, a0_ref, a1_ref, wn1_ref, wn2_ref, bn_ref, wnp_ref,
               bnp_ref, out_ref):
    agg = a0_ref[...] + a1_ref[...]
    h = (
        jnp.dot(x_ref[...], wn1_ref[...], preferred_element_type=jnp.float32)
        + jnp.dot(agg, wn2_ref[...], preferred_element_type=jnp.float32)
        + bn_ref[...]
    )
    h = jnp.maximum(h, 0.0)
    out_ref[...] = (
        jnp.dot(h, wnp_ref[...], preferred_element_type=jnp.float32)
        + bnp_ref[...]
    )


def _node_mlp(x, a0, a1, wn1, wn2, b_node, W_np, b_np):
    N, D = x.shape
    H = wn1.shape[1]
    NCo = W_np.shape[1]
    NB = 2000
    return pl.pallas_call(
        _node_body,
        grid=(N // NB,),
        in_specs=[
            pl.BlockSpec((NB, D), lambda i: (i, 0)),
            pl.BlockSpec((NB, D), lambda i: (i, 0)),
            pl.BlockSpec((NB, D), lambda i: (i, 0)),
            pl.BlockSpec((D, H), lambda i: (0, 0)),
            pl.BlockSpec((D, H), lambda i: (0, 0)),
            pl.BlockSpec((1, H), lambda i: (0, 0)),
            pl.BlockSpec((H, NCo), lambda i: (0, 0)),
            pl.BlockSpec((1, NCo), lambda i: (0, 0)),
        ],
        out_specs=pl.BlockSpec((NB, NCo), lambda i: (i, 0)),
        out_shape=jax.ShapeDtypeStruct((N, NCo), jnp.float32),
    )(x, a0, a1, wn1, wn2, b_node.reshape(1, H), W_np, b_np.reshape(1, NCo))


def kernel(x, edge_index, edge_attr, batch, W_gate, b_gate, W_node, b_node,
           W_np, b_np, W_ep, b_ep):
    N, D = x.shape
    src = edge_index[0]
    dst = edge_index[1]

    gate = _compute_gate(edge_attr, W_gate, b_gate)
    wep2 = W_ep[D:]
    y2 = _compute_y2(x, wep2, b_ep)
    wep1 = W_ep[:D].T.reshape(2, D)

    ep0v, ep1v, aggp = _sc_edge_kernel(x, src, dst, gate, y2, wep1)

    wn1 = W_node[:D]
    wn2 = W_node[D:]
    node_pred = _node_mlp(x, aggp[0], aggp[1], wn1, wn2, b_node, W_np, b_np)
    edge_pred = jnp.stack([ep0v, ep1v], axis=1)
    return node_pred, edge_pred


# speedup vs baseline: 1.6952x; 1.0796x over previous
import functools

import jax
import jax.numpy as jnp
from jax import lax
from jax.experimental import pallas as pl
from jax.experimental.pallas import tpu as pltpu
from jax.experimental.pallas import tpu_sc as plsc


def _gate_body(ea_ref, wg_ref, bg_ref, out_ref):
    z = jnp.dot(ea_ref[...], wg_ref[...], preferred_element_type=jnp.float32)
    out_ref[...] = jax.nn.sigmoid(z + bg_ref[...])


def _compute_gate(edge_attr, W_gate, b_gate):
    E, DE = edge_attr.shape
    D = W_gate.shape[1]
    EB = E // 40
    return pl.pallas_call(
        _gate_body,
        grid=(E // EB,),
        in_specs=[
            pl.BlockSpec((EB, DE), lambda i: (i, 0)),
            pl.BlockSpec((DE, D), lambda i: (0, 0)),
            pl.BlockSpec((1, D), lambda i: (0, 0)),
        ],
        out_specs=pl.BlockSpec((EB, D), lambda i: (i, 0)),
        out_shape=jax.ShapeDtypeStruct((E, D), jnp.float32),
    )(edge_attr, W_gate, b_gate.reshape(1, D))


def _y2_body(x_ref, w_ref, b_ref, out_ref):
    out_ref[...] = (
        jnp.dot(x_ref[...], w_ref[...], preferred_element_type=jnp.float32)
        + b_ref[...]
    )


def _compute_y2(x, wep2, b_ep):
    N, D = x.shape
    EC = wep2.shape[1]
    w = jnp.pad(wep2, ((0, 0), (0, 16 - EC)))
    b = jnp.pad(b_ep, (0, 16 - EC)).reshape(1, 16)
    NB = 2000
    return pl.pallas_call(
        _y2_body,
        grid=(N // NB,),
        in_specs=[
            pl.BlockSpec((NB, D), lambda i: (i, 0)),
            pl.BlockSpec((D, 16), lambda i: (0, 0)),
            pl.BlockSpec((1, 16), lambda i: (0, 0)),
        ],
        out_specs=pl.BlockSpec((NB, 16), lambda i: (i, 0)),
        out_shape=jax.ShapeDtypeStruct((N, 16), jnp.float32),
    )(x, w, b)


def _vshuffle(x, idx16):
    return jax.lax.gather(
        x,
        idx16[:, None],
        jax.lax.GatherDimensionNumbers(
            offset_dims=(), collapsed_slice_dims=(0,), start_index_map=(0,)
        ),
        slice_sizes=(1,),
        mode=jax.lax.GatherScatterMode.PROMISE_IN_BOUNDS,
    )

_NCORES = 2
_NSUB = 16
_NW = _NCORES * _NSUB
_L = 16
_CHUNK = 80
_NBUF = 2


def _sc_edge_kernel(x, src, dst, gate, y2, wep1):
    N, D = x.shape
    E = src.shape[0]
    epw = E // _NW
    nch = epw // _CHUNK
    ngr = _CHUNK // _L
    nk = D // _L
    npad = ((N + _NSUB * 8 - 1) // (_NSUB * 8)) * (_NSUB * 8)
    rpt = npad // _NSUB
    ntr = nch // _NBUF
    tail = nch % _NBUF
    fl = max(d for d in range(1, 33) if nch % d == 0)

    mesh = plsc.VectorSubcoreMesh(
        core_axis_name="c", subcore_axis_name="s",
        num_cores=_NCORES, num_subcores=_NSUB,
    )

    @functools.partial(
        pl.kernel,
        out_type=[
            jax.ShapeDtypeStruct((E,), jnp.float32),
            jax.ShapeDtypeStruct((E,), jnp.float32),
            jax.ShapeDtypeStruct((_NCORES, npad, D), jnp.float32),
        ],
        mesh=mesh,
        compiler_params=pltpu.CompilerParams(use_tc_tiling_on_sc=False),
        scratch_types=[
            [pltpu.VMEM((2, _CHUNK), jnp.int32)] * _NBUF,
            [pltpu.VMEM((_CHUNK, D), jnp.float32)] * _NBUF,
            [pltpu.VMEM((_CHUNK, D), jnp.float32)] * _NBUF,
            pltpu.VMEM((fl * _CHUNK,), jnp.float32),
            pltpu.VMEM((fl * _CHUNK,), jnp.float32),
            [pltpu.VMEM((_CHUNK, 16), jnp.float32)] * _NBUF,
            pltpu.VMEM((2, D), jnp.float32),
            pltpu.VMEM_SHARED((npad, D), jnp.float32),
            [pltpu.SemaphoreType.DMA] * _NBUF,
            [pltpu.SemaphoreType.DMA] * _NBUF,
        ],
    )
    def run(x_hbm, sd_hbm, gate_hbm, y2_hbm, wep1_hbm, zeros_hbm,
            ep0_hbm, ep1_hbm, aggp_hbm,
            sdidx, rows, gatev, ep0, ep1, y2r, w_v, agg_sh,
            insem, outsem):
        cid = lax.axis_index("c")
        sid = lax.axis_index("s")
        wid = cid * _NSUB + sid

        pltpu.sync_copy(wep1_hbm, w_v)

        zf = jnp.zeros((_L,), jnp.float32)
        pltpu.sync_copy(zeros_hbm, agg_sh.at[pl.ds(sid * rpt, rpt)])
        plsc.subcore_barrier()

        iota16 = lax.iota(jnp.int32, _L)
        zi = jnp.zeros((_L,), jnp.int32)
        oi = zi + 1
        bfly = [iota16 ^ s for s in (8, 4, 2, 1)]
        wa = [w_v[0, pl.ds(k * _L, _L)] for k in range(nk)]
        wb = [w_v[1, pl.ds(k * _L, _L)] for k in range(nk)]

        def issue_ins(i, s):
            base = wid * epw + i * _CHUNK
            pltpu.sync_copy(sd_hbm.at[wid * nch + i], sdidx[s])
            pltpu.async_copy(x_hbm.at[sdidx[s].at[0]], rows[s], insem[s])
            pltpu.async_copy(y2_hbm.at[sdidx[s].at[1]], y2r[s], insem[s])
            pltpu.async_copy(gate_hbm.at[pl.ds(base, _CHUNK)], gatev[s],
                             insem[s])

        def drain_ins(i, s):
            base = wid * epw + i * _CHUNK
            pltpu.make_async_copy(x_hbm.at[sdidx[s].at[0]], rows[s],
                                  insem[s]).wait()
            pltpu.make_async_copy(y2_hbm.at[sdidx[s].at[1]], y2r[s],
                                  insem[s]).wait()
            pltpu.make_async_copy(gate_hbm.at[pl.ds(base, _CHUNK)], gatev[s],
                                  insem[s]).wait()

        def issue_outs(i, s):
            pltpu.sync_copy(rows[s], agg_sh.at[sdidx[s].at[1]], add=True)

            @pl.when(lax.rem(i, fl) == fl - 1)
            def _():
                fbase = wid * epw + (i - (fl - 1)) * _CHUNK
                pltpu.sync_copy(ep0, ep0_hbm.at[pl.ds(fbase, fl * _CHUNK)])
                pltpu.sync_copy(ep1, ep1_hbm.at[pl.ds(fbase, fl * _CHUNK)])

        def compute(i, s):
            eoff = lax.rem(i, fl) * _CHUNK

            def group(g, _):
                ep0acc = zf
                ep1acc = zf
                for j in range(_L):
                    e = g * _L + j
                    acc0 = zf
                    acc1 = zf
                    for k in range(nk):
                        xv = rows[s][e, pl.ds(k * _L, _L)]
                        gv = gatev[s][e, pl.ds(k * _L, _L)]
                        mv = xv * gv
                        rows[s][e, pl.ds(k * _L, _L)] = mv
                        acc0 = acc0 + mv * wa[k]
                        acc1 = acc1 + mv * wb[k]
                    for st in bfly:
                        acc0 = acc0 + _vshuffle(acc0, st)
                        acc1 = acc1 + _vshuffle(acc1, st)
                    yrow = y2r[s][e, :]
                    sel = iota16 == j
                    ep0acc = jnp.where(sel, acc0 + _vshuffle(yrow, zi), ep0acc)
                    ep1acc = jnp.where(sel, acc1 + _vshuffle(yrow, oi), ep1acc)
                ep0[pl.ds(eoff + g * _L, _L)] = ep0acc
                ep1[pl.ds(eoff + g * _L, _L)] = ep1acc
                return 0

            lax.fori_loop(0, ngr, group, 0)

        def phase(i, s):
            drain_ins(i, s)
            compute(i, s)
            issue_outs(i, s)

            @pl.when(i + _NBUF < nch)
            def _():
                issue_ins(i + _NBUF, s)

        issue_ins(0, 0)
        issue_ins(1, 1)

        def tbody(t, _):
            i0 = t * _NBUF
            phase(i0, 0)
            phase(i0 + 1, 1)
            return 0

        lax.fori_loop(0, ntr, tbody, 0)

        if tail:
            phase(nch - 1, (nch - 1) % _NBUF)

        plsc.subcore_barrier()

        pltpu.sync_copy(
            agg_sh.at[pl.ds(sid * rpt, rpt)],
            aggp_hbm.at[cid, pl.ds(sid * rpt, rpt)],
        )

    sd = jnp.stack(
        [src.reshape(_NW * nch, _CHUNK), dst.reshape(_NW * nch, _CHUNK)],
        axis=1,
    )
    zeros = jnp.zeros((rpt, D), jnp.float32)
    return run(x, sd, gate, y2, wep1, zeros)


def _node_body(x_ref, a0_ref, a1_ref, a2_ref, a3_ref, wn1_ref, wn2_ref,
               bn_ref, wnp_ref, bnp_ref, out_ref):
    agg = (a0_ref[...] + a1_ref[...]) + (a2_ref[...] + a3_ref[...])
    h = (
        jnp.dot(x_ref[...], wn1_ref[...], preferred_element_type=jnp.float32)
        + jnp.dot(agg, wn2_ref[...], preferred_element_type=jnp.float32)
        + bn_ref[...]
    )
    h = jnp.maximum(h, 0.0)
    out_ref[...] = (
        jnp.dot(h, wnp_ref[...], preferred_element_type=jnp.float32)
        + bnp_ref[...]
    )


def _node_mlp(x, a0, a1, a2, a3, wn1, wn2, b_node, W_np, b_np):
    N, D = x.shape
    H = wn1.shape[1]
    NCo = W_np.shape[1]
    NB = 2000
    return pl.pallas_call(
        _node_body,
        grid=(N // NB,),
        in_specs=[
            pl.BlockSpec((NB, D), lambda i: (i, 0)),
            pl.BlockSpec((NB, D), lambda i: (i, 0)),
            pl.BlockSpec((NB, D), lambda i: (i, 0)),
            pl.BlockSpec((NB, D), lambda i: (i, 0)),
            pl.BlockSpec((NB, D), lambda i: (i, 0)),
            pl.BlockSpec((D, H), lambda i: (0, 0)),
            pl.BlockSpec((D, H), lambda i: (0, 0)),
            pl.BlockSpec((1, H), lambda i: (0, 0)),
            pl.BlockSpec((H, NCo), lambda i: (0, 0)),
            pl.BlockSpec((1, NCo), lambda i: (0, 0)),
        ],
        out_specs=pl.BlockSpec((NB, NCo), lambda i: (i, 0)),
        out_shape=jax.ShapeDtypeStruct((N, NCo), jnp.float32),
    )(x, a0, a1, a2, a3, wn1, wn2, b_node.reshape(1, H), W_np,
      b_np.reshape(1, NCo))


def kernel(x, edge_index, edge_attr, batch, W_gate, b_gate, W_node, b_node,
           W_np, b_np, W_ep, b_ep):
    N, D = x.shape
    E = edge_index.shape[1]
    src = edge_index[0]
    dst = edge_index[1]

    wep2 = W_ep[D:]
    y2 = _compute_y2(x, wep2, b_ep)
    wep1 = W_ep[:D].T.reshape(2, D)

    e1 = (E // 2 // (_NW * _CHUNK)) * (_NW * _CHUNK) + _NW * _CHUNK
    halves = []
    for lo, hi in ((0, e1), (e1, E)):
        gate_h = _compute_gate(edge_attr[lo:hi], W_gate, b_gate)
        halves.append(
            _sc_edge_kernel(x, src[lo:hi], dst[lo:hi], gate_h, y2, wep1)
        )
    (ep0a, ep1a, aggpa), (ep0b, ep1b, aggpb) = halves

    wn1 = W_node[:D]
    wn2 = W_node[D:]
    node_pred = _node_mlp(x, aggpa[0], aggpa[1], aggpb[0], aggpb[1],
                          wn1, wn2, b_node, W_np, b_np)
    edge_pred = jnp.stack(
        [jnp.concatenate([ep0a, ep0b]), jnp.concatenate([ep1a, ep1b])],
        axis=1,
    )
    return node_pred, edge_pred
